# baseline (speedup 1.0000x reference)
import jax
import jax.numpy as jnp
from jax.experimental import pallas as pl

_BM = 2000


def _mlp_kernel(x_ref, w1_ref, b1_ref, w2_ref, b2_ref, o_ref):
    x = x_ref[...]
    h = jnp.dot(x, w1_ref[...], preferred_element_type=jnp.float32)
    h = jnp.maximum(h + b1_ref[...], 0.0)
    o = jnp.dot(h, w2_ref[...], preferred_element_type=jnp.float32)
    o_ref[...] = o + b2_ref[...]


def kernel(features, W1, b1, W2, b2):
    n, d = features.shape
    h = W1.shape[1]
    c = W2.shape[1]
    return pl.pallas_call(
        _mlp_kernel,
        grid=(n // _BM,),
        in_specs=[
            pl.BlockSpec((_BM, d), lambda i: (i, 0)),
            pl.BlockSpec((d, h), lambda i: (0, 0)),
            pl.BlockSpec((1, h), lambda i: (0, 0)),
            pl.BlockSpec((d, c), lambda i: (0, 0)),
            pl.BlockSpec((1, c), lambda i: (0, 0)),
        ],
        out_specs=pl.BlockSpec((_BM, c), lambda i: (i, 0)),
        out_shape=jax.ShapeDtypeStruct((n, c), jnp.float32),
    )(features, W1, b1.reshape(1, h), W2, b2.reshape(1, c))


# speedup vs baseline: 1.0019x; 1.0019x over previous
import jax
import jax.numpy as jnp
from jax.experimental import pallas as pl
from jax.experimental.pallas import tpu as pltpu

_BM = 2000


def _mlp_kernel(x_ref, w1_ref, b1_ref, w2_ref, b2_ref, o_ref):
    x = x_ref[...]
    h = jnp.dot(x, w1_ref[...], preferred_element_type=jnp.float32)
    h = jnp.maximum(h + b1_ref[...], 0.0)
    o = jnp.dot(h, w2_ref[...], preferred_element_type=jnp.float32)
    o_ref[...] = o + b2_ref[...]


def kernel(features, W1, b1, W2, b2):
    n, d = features.shape
    h = W1.shape[1]
    c = W2.shape[1]
    return pl.pallas_call(
        _mlp_kernel,
        grid=(n // _BM,),
        in_specs=[
            pl.BlockSpec((_BM, d), lambda i: (i, 0)),
            pl.BlockSpec((d, h), lambda i: (0, 0)),
            pl.BlockSpec((1, h), lambda i: (0, 0)),
            pl.BlockSpec((d, c), lambda i: (0, 0)),
            pl.BlockSpec((1, c), lambda i: (0, 0)),
        ],
        out_specs=pl.BlockSpec((_BM, c), lambda i: (i, 0)),
        out_shape=jax.ShapeDtypeStruct((n, c), jnp.float32),
        compiler_params=pltpu.CompilerParams(
            dimension_semantics=("parallel",),
        ),
    )(features, W1, b1.reshape(1, h), W2, b2.reshape(1, c))


# speedup vs baseline: 1.6087x; 1.6057x over previous
import jax
import jax.numpy as jnp
from jax import lax
from jax.experimental import pallas as pl
from jax.experimental.pallas import tpu as pltpu

_BM = 2048


def _mlp_kernel(x_ref, w1_ref, b1_ref, w2t_ref, b2_ref, ot_ref):
    x = x_ref[...].astype(jnp.bfloat16)
    w1 = w1_ref[...].astype(jnp.bfloat16)
    h = jnp.dot(x, w1, preferred_element_type=jnp.float32)
    h = jnp.maximum(h + b1_ref[...], 0.0).astype(jnp.bfloat16)
    w2t = w2t_ref[...].astype(jnp.bfloat16)
    ot = lax.dot_general(w2t, h, (((1,), (1,)), ((), ())),
                         preferred_element_type=jnp.float32)
    ot_ref[...] = ot + b2_ref[...]


def kernel(features, W1, b1, W2, b2):
    n, d = features.shape
    h = W1.shape[1]
    c = W2.shape[1]
    out_t = pl.pallas_call(
        _mlp_kernel,
        grid=(pl.cdiv(n, _BM),),
        in_specs=[
            pl.BlockSpec((_BM, d), lambda i: (i, 0)),
            pl.BlockSpec((d, h), lambda i: (0, 0)),
            pl.BlockSpec((1, h), lambda i: (0, 0)),
            pl.BlockSpec((c, d), lambda i: (0, 0)),
            pl.BlockSpec((c, 1), lambda i: (0, 0)),
        ],
        out_specs=pl.BlockSpec((c, _BM), lambda i: (0, i)),
        out_shape=jax.ShapeDtypeStruct((c, n), jnp.float32),
        compiler_params=pltpu.CompilerParams(
            dimension_semantics=("parallel",),
        ),
    )(features, W1, b1.reshape(1, h), W2.T, b2.reshape(c, 1))
    return out_t.T


# speedup vs baseline: 1.9847x; 1.2337x over previous
import jax
import jax.numpy as jnp
from jax import lax
from jax.experimental import pallas as pl
from jax.experimental.pallas import tpu as pltpu

_BM = 4096


def _mlp_kernel(x_ref, w1_ref, b1_ref, w2t_ref, b2_ref, ot_ref):
    x = x_ref[...].astype(jnp.bfloat16)
    w1 = w1_ref[...].astype(jnp.bfloat16)
    h = jnp.dot(x, w1, preferred_element_type=jnp.float32)
    h = jnp.maximum(h + b1_ref[...], 0.0).astype(jnp.bfloat16)
    w2t = w2t_ref[...].astype(jnp.bfloat16)
    ot = lax.dot_general(w2t, h, (((1,), (1,)), ((), ())),
                         preferred_element_type=jnp.float32)
    ot_ref[...] = ot + b2_ref[...]


def kernel(features, W1, b1, W2, b2):
    n, d = features.shape
    h = W1.shape[1]
    c = W2.shape[1]
    out_t = pl.pallas_call(
        _mlp_kernel,
        grid=(pl.cdiv(n, _BM),),
        in_specs=[
            pl.BlockSpec((_BM, d), lambda i: (i, 0)),
            pl.BlockSpec((d, h), lambda i: (0, 0)),
            pl.BlockSpec((1, h), lambda i: (0, 0)),
            pl.BlockSpec((c, d), lambda i: (0, 0)),
            pl.BlockSpec((c, 1), lambda i: (0, 0)),
        ],
        out_specs=pl.BlockSpec((c, _BM), lambda i: (0, i)),
        out_shape=jax.ShapeDtypeStruct((c, n), jnp.float32),
        compiler_params=pltpu.CompilerParams(
            dimension_semantics=("parallel",),
        ),
    )(features, W1, b1.reshape(1, h), W2.T, b2.reshape(c, 1))
    return out_t.T


# speedup vs baseline: 2.1524x; 1.0845x over previous
import jax
import jax.numpy as jnp
from jax import lax
from jax.experimental import pallas as pl
from jax.experimental.pallas import tpu as pltpu

_BM = 8192


def _mlp_kernel(x_ref, w1_ref, b1_ref, w2t_ref, b2_ref, ot_ref):
    x = x_ref[...].astype(jnp.bfloat16)
    w1 = w1_ref[...].astype(jnp.bfloat16)
    h = jnp.dot(x, w1, preferred_element_type=jnp.float32)
    h = jnp.maximum(h + b1_ref[...], 0.0).astype(jnp.bfloat16)
    w2t = w2t_ref[...].astype(jnp.bfloat16)
    ot = lax.dot_general(w2t, h, (((1,), (1,)), ((), ())),
                         preferred_element_type=jnp.float32)
    ot_ref[...] = ot + b2_ref[...]


def kernel(features, W1, b1, W2, b2):
    n, d = features.shape
    h = W1.shape[1]
    c = W2.shape[1]
    out_t = pl.pallas_call(
        _mlp_kernel,
        grid=(pl.cdiv(n, _BM),),
        in_specs=[
            pl.BlockSpec((_BM, d), lambda i: (i, 0)),
            pl.BlockSpec((d, h), lambda i: (0, 0)),
            pl.BlockSpec((1, h), lambda i: (0, 0)),
            pl.BlockSpec((c, d), lambda i: (0, 0)),
            pl.BlockSpec((c, 1), lambda i: (0, 0)),
        ],
        out_specs=pl.BlockSpec((c, _BM), lambda i: (0, i)),
        out_shape=jax.ShapeDtypeStruct((c, n), jnp.float32),
        compiler_params=pltpu.CompilerParams(
            dimension_semantics=("parallel",),
        ),
    )(features, W1, b1.reshape(1, h), W2.T, b2.reshape(c, 1))
    return out_t.T


# speedup vs baseline: 2.2997x; 1.0684x over previous
import jax
import jax.numpy as jnp
from jax import lax
from jax.experimental import pallas as pl
from jax.experimental.pallas import tpu as pltpu

_BM = 12800


def _mlp_kernel(x_ref, w1_ref, b1_ref, w2t_ref, b2_ref, ot_ref):
    x = x_ref[...].astype(jnp.bfloat16)
    w1 = w1_ref[...].astype(jnp.bfloat16)
    h = jnp.dot(x, w1, preferred_element_type=jnp.float32)
    h = jnp.maximum(h + b1_ref[...], 0.0).astype(jnp.bfloat16)
    w2t = w2t_ref[...].astype(jnp.bfloat16)
    ot = lax.dot_general(w2t, h, (((1,), (1,)), ((), ())),
                         preferred_element_type=jnp.float32)
    ot_ref[...] = ot + b2_ref[...]


def kernel(features, W1, b1, W2, b2):
    n, d = features.shape
    h = W1.shape[1]
    c = W2.shape[1]
    out_t = pl.pallas_call(
        _mlp_kernel,
        grid=(pl.cdiv(n, _BM),),
        in_specs=[
            pl.BlockSpec((_BM, d), lambda i: (i, 0)),
            pl.BlockSpec((d, h), lambda i: (0, 0)),
            pl.BlockSpec((1, h), lambda i: (0, 0)),
            pl.BlockSpec((c, d), lambda i: (0, 0)),
            pl.BlockSpec((c, 1), lambda i: (0, 0)),
        ],
        out_specs=pl.BlockSpec((c, _BM), lambda i: (0, i)),
        out_shape=jax.ShapeDtypeStruct((c, n), jnp.float32),
        compiler_params=pltpu.CompilerParams(
            dimension_semantics=("parallel",),
        ),
    )(features, W1, b1.reshape(1, h), W2.T, b2.reshape(c, 1))
    return out_t.T
